# Initial kernel scaffold; baseline (speedup 1.0000x reference)
import jax
import jax.numpy as jnp
from jax.experimental import pallas as pl


def kernel(x, edge_index, W1_0, b1_0, bng_0, bnb_0, p1_0, W2_0, b2_0, lng_0, lnb_0, p2_0, W1_1, b1_1, bng_1, bnb_1, p1_1, W2_1, b2_1, lng_1, lnb_1, p2_1):
    raise NotImplementedError("write your pallas kernel here")


# speedup vs baseline: 4.5630x; 4.5630x over previous
import functools

import jax
import jax.numpy as jnp
from jax import lax
from jax.experimental import pallas as pl
from jax.experimental.pallas import tpu as pltpu
from jax.experimental.pallas import tpu_sc as plsc

N = 10000
E = 320000
D = 128

NC = 2
NS = 16
NW = NC * NS

EDGES_PER_TILE = E // NW
CHUNK = 80
NCHUNKS = EDGES_PER_TILE // CHUNK
ROWS_PER_TILE = 624
TAIL_ROW0 = NS * ROWS_PER_TILE
TAIL_ROWS = N - TAIL_ROW0


def _sc_agg_body(h_hbm, src_hbm, dst_hbm, zeros_hbm, out_hbm,
                 src_v, dst_v, rows_v, agg_s, sem):
    c = lax.axis_index("c")
    s = lax.axis_index("s")

    row0 = s * ROWS_PER_TILE
    pltpu.sync_copy(zeros_hbm.at[pl.ds(row0, ROWS_PER_TILE)],
                    agg_s.at[pl.ds(row0, ROWS_PER_TILE)])

    @pl.when(s == NS - 1)
    def _():
        pltpu.sync_copy(zeros_hbm.at[pl.ds(TAIL_ROW0, TAIL_ROWS)],
                        agg_s.at[pl.ds(TAIL_ROW0, TAIL_ROWS)])

    plsc.subcore_barrier()

    base = (c * NS + s) * EDGES_PER_TILE

    def chunk(i, carry):
        off = base + i * CHUNK
        pltpu.sync_copy(src_hbm.at[pl.ds(off, CHUNK)], src_v)
        pltpu.sync_copy(dst_hbm.at[pl.ds(off, CHUNK)], dst_v)
        pltpu.async_copy(h_hbm.at[src_v], rows_v, sem).wait()
        pltpu.sync_copy(rows_v, agg_s.at[dst_v], add=True)
        return carry

    lax.fori_loop(0, NCHUNKS, chunk, 0)
    plsc.subcore_barrier()

    pltpu.sync_copy(agg_s.at[pl.ds(row0, ROWS_PER_TILE)],
                    out_hbm.at[c, pl.ds(row0, ROWS_PER_TILE)])

    @pl.when(s == NS - 1)
    def _():
        pltpu.sync_copy(agg_s.at[pl.ds(TAIL_ROW0, TAIL_ROWS)],
                        out_hbm.at[c, pl.ds(TAIL_ROW0, TAIL_ROWS)])


_sc_agg = functools.partial(
    pl.kernel,
    out_type=jax.ShapeDtypeStruct((NC, N, D), jnp.float32),
    mesh=plsc.VectorSubcoreMesh(core_axis_name="c", subcore_axis_name="s"),
    scratch_types=[
        pltpu.VMEM((CHUNK,), jnp.int32),
        pltpu.VMEM((CHUNK,), jnp.int32),
        pltpu.VMEM((CHUNK, D), jnp.float32),
        pltpu.VMEM_SHARED((N, D), jnp.float32),
        pltpu.SemaphoreType.DMA,
    ],
)(_sc_agg_body)


def _mlp_body(h_ref, a0_ref, a1_ref,
              W1_ref, b1_ref, bng_ref, bnb_ref, p1_ref,
              W2_ref, b2_ref, lng_ref, lnb_ref, p2_ref, out_ref):
    z = h_ref[...] + a0_ref[...] + a1_ref[...]
    z = jnp.dot(z, W1_ref[...], preferred_element_type=jnp.float32) + b1_ref[...]
    mu = jnp.mean(z, axis=0, keepdims=True)
    var = jnp.mean((z - mu) ** 2, axis=0, keepdims=True)
    z = (z - mu) * lax.rsqrt(var + 1e-5) * bng_ref[...] + bnb_ref[...]
    z = jnp.where(z >= 0, z, p1_ref[0, 0] * z)
    z = jnp.dot(z, W2_ref[...], preferred_element_type=jnp.float32) + b2_ref[...]
    lmu = jnp.mean(z, axis=1, keepdims=True)
    lvar = jnp.mean((z - lmu) ** 2, axis=1, keepdims=True)
    z = (z - lmu) * lax.rsqrt(lvar + 1e-5) * lng_ref[...] + lnb_ref[...]
    out_ref[...] = jnp.where(z >= 0, z, p2_ref[0, 0] * z)


def _mlp(h, a0, a1, W1, b1, bng, bnb, p1, W2, b2, lng, lnb, p2):
    return pl.pallas_call(
        _mlp_body,
        out_shape=jax.ShapeDtypeStruct((N, D), jnp.float32),
    )(h, a0, a1,
      W1, b1.reshape(1, D), bng.reshape(1, D), bnb.reshape(1, D),
      p1.reshape(1, 1),
      W2, b2.reshape(1, D), lng.reshape(1, D), lnb.reshape(1, D),
      p2.reshape(1, 1))


def kernel(x, edge_index, W1_0, b1_0, bng_0, bnb_0, p1_0, W2_0, b2_0, lng_0,
           lnb_0, p2_0, W1_1, b1_1, bng_1, bnb_1, p1_1, W2_1, b2_1, lng_1,
           lnb_1, p2_1):
    src = edge_index[0]
    dst = edge_index[1]
    zeros = jnp.zeros((N, D), jnp.float32)

    parts = _sc_agg(x, src, dst, zeros)
    h1 = _mlp(x, parts[0], parts[1],
              W1_0, b1_0, bng_0, bnb_0, p1_0, W2_0, b2_0, lng_0, lnb_0, p2_0)
    parts = _sc_agg(h1, src, dst, zeros)
    h2 = _mlp(h1, parts[0], parts[1],
              W1_1, b1_1, bng_1, bnb_1, p1_1, W2_1, b2_1, lng_1, lnb_1, p2_1)
    return h2


# speedup vs baseline: 7.4107x; 1.6241x over previous
import functools

import jax
import jax.numpy as jnp
from jax import lax
from jax.experimental import pallas as pl
from jax.experimental.pallas import tpu as pltpu
from jax.experimental.pallas import tpu_sc as plsc

N = 10000
E = 320000
D = 128
DH = D // 2
NP = N + 8

NC = 2
NS = 16

CHUNK = 128
EPT = E // NS
EPT_PAD = 20480
NCHUNKS = EPT_PAD // CHUNK
PAD_PER_TILE = EPT_PAD - EPT
ROWS_PER_TILE = 624
TAIL_ROW0 = NS * ROWS_PER_TILE
TAIL_ROWS = N - TAIL_ROW0


def _sc_agg_body(h_hbm, src_hbm, dst_hbm, zeros_hbm, out_hbm,
                 src_v, dst_v, r0, r1, r2, r3, agg_s, m0, m1, m2, m3):
    c = lax.axis_index("c")
    s = lax.axis_index("s")
    rows = (r0, r1, r2, r3)
    sems = (m0, m1, m2, m3)

    ebase = s * EPT_PAD
    pltpu.sync_copy(src_hbm.at[pl.ds(ebase, EPT_PAD)], src_v)
    pltpu.sync_copy(dst_hbm.at[pl.ds(ebase, EPT_PAD)], dst_v)

    row0 = s * ROWS_PER_TILE
    pltpu.sync_copy(zeros_hbm.at[pl.ds(row0, ROWS_PER_TILE)],
                    agg_s.at[pl.ds(row0, ROWS_PER_TILE)])

    @pl.when(s == NS - 1)
    def _():
        pltpu.sync_copy(zeros_hbm.at[pl.ds(TAIL_ROW0, TAIL_ROWS)],
                        agg_s.at[pl.ds(TAIL_ROW0, TAIL_ROWS)])

    plsc.subcore_barrier()

    def sidx(k):
        return src_v.at[pl.ds(k * CHUNK, CHUNK)]

    def didx(k):
        return dst_v.at[pl.ds(k * CHUNK, CHUNK)]

    def issue_gather(k, b):
        pltpu.async_copy(h_hbm.at[c].at[sidx(k)], rows[b], sems[b])

    def wait_gather(k, b):
        pltpu.make_async_copy(h_hbm.at[c].at[sidx(k)], rows[b], sems[b]).wait()

    def issue_scatter(k, b):
        pltpu.async_copy(rows[b], agg_s.at[didx(k)], sems[b], add=True)

    def wait_scatter(k, b):
        pltpu.make_async_copy(rows[b], agg_s.at[didx(k)], sems[b]).wait()

    issue_gather(0, 0)
    issue_gather(1, 1)
    wait_gather(0, 0)
    issue_scatter(0, 0)
    issue_gather(2, 2)
    wait_gather(1, 1)
    issue_scatter(1, 1)
    issue_gather(3, 3)

    def body(i, carry):
        for j in range(4):
            k = 2 + 4 * i + j
            b = (2 + j) % 4
            wait_gather(k, b)
            issue_scatter(k, b)
            wait_scatter(k - 2, j)
            issue_gather(k + 2, j)
        return carry

    lax.fori_loop(0, (NCHUNKS - 4) // 4, body, 0)

    k = NCHUNKS - 2
    wait_gather(k, 2)
    issue_scatter(k, 2)
    wait_scatter(k - 2, 0)
    wait_gather(k + 1, 3)
    issue_scatter(k + 1, 3)
    wait_scatter(k - 1, 1)
    wait_scatter(k, 2)
    wait_scatter(k + 1, 3)

    plsc.subcore_barrier()

    pltpu.sync_copy(agg_s.at[pl.ds(row0, ROWS_PER_TILE)],
                    out_hbm.at[c, pl.ds(row0, ROWS_PER_TILE)])

    @pl.when(s == NS - 1)
    def _():
        pltpu.sync_copy(agg_s.at[pl.ds(TAIL_ROW0, TAIL_ROWS)],
                        out_hbm.at[c, pl.ds(TAIL_ROW0, TAIL_ROWS)])


_sc_agg = functools.partial(
    pl.kernel,
    out_type=jax.ShapeDtypeStruct((NC, N, DH), jnp.float32),
    mesh=plsc.VectorSubcoreMesh(core_axis_name="c", subcore_axis_name="s"),
    compiler_params=pltpu.CompilerParams(use_tc_tiling_on_sc=False),
    scratch_types=[
        pltpu.VMEM((EPT_PAD,), jnp.int32),
        pltpu.VMEM((EPT_PAD,), jnp.int32),
        pltpu.VMEM((CHUNK, DH), jnp.float32),
        pltpu.VMEM((CHUNK, DH), jnp.float32),
        pltpu.VMEM((CHUNK, DH), jnp.float32),
        pltpu.VMEM((CHUNK, DH), jnp.float32),
        pltpu.VMEM_SHARED((N, DH), jnp.float32),
        pltpu.SemaphoreType.DMA,
        pltpu.SemaphoreType.DMA,
        pltpu.SemaphoreType.DMA,
        pltpu.SemaphoreType.DMA,
    ],
)(_sc_agg_body)


def _mlp_core(h_ref, a_ref, W1_ref, b1_ref, bng_ref, bnb_ref, p1_ref,
              W2_ref, b2_ref, lng_ref, lnb_ref, p2_ref):
    z0 = h_ref[0, :N, :] + a_ref[0]
    z1 = h_ref[1, :N, :] + a_ref[1]
    z = (jnp.dot(z0, W1_ref[:DH, :], preferred_element_type=jnp.float32)
         + jnp.dot(z1, W1_ref[DH:, :], preferred_element_type=jnp.float32)
         + b1_ref[...])
    mu = jnp.mean(z, axis=0, keepdims=True)
    var = jnp.mean((z - mu) ** 2, axis=0, keepdims=True)
    z = (z - mu) * lax.rsqrt(var + 1e-5) * bng_ref[...] + bnb_ref[...]
    z = jnp.where(z >= 0, z, p1_ref[0, 0] * z)
    z = jnp.dot(z, W2_ref[...], preferred_element_type=jnp.float32) + b2_ref[...]
    lmu = jnp.mean(z, axis=1, keepdims=True)
    lvar = jnp.mean((z - lmu) ** 2, axis=1, keepdims=True)
    z = (z - lmu) * lax.rsqrt(lvar + 1e-5) * lng_ref[...] + lnb_ref[...]
    return jnp.where(z >= 0, z, p2_ref[0, 0] * z)


def _mlp_mid_body(h_ref, a_ref, W1_ref, b1_ref, bng_ref, bnb_ref, p1_ref,
                  W2_ref, b2_ref, lng_ref, lnb_ref, p2_ref, out_ref):
    y = _mlp_core(h_ref, a_ref, W1_ref, b1_ref, bng_ref, bnb_ref, p1_ref,
                  W2_ref, b2_ref, lng_ref, lnb_ref, p2_ref)
    out_ref[0, :N, :] = y[:, :DH]
    out_ref[1, :N, :] = y[:, DH:]
    out_ref[0, N:, :] = jnp.zeros((NP - N, DH), jnp.float32)
    out_ref[1, N:, :] = jnp.zeros((NP - N, DH), jnp.float32)


def _mlp_fin_body(h_ref, a_ref, W1_ref, b1_ref, bng_ref, bnb_ref, p1_ref,
                  W2_ref, b2_ref, lng_ref, lnb_ref, p2_ref, out_ref):
    out_ref[...] = _mlp_core(h_ref, a_ref, W1_ref, b1_ref, bng_ref, bnb_ref,
                             p1_ref, W2_ref, b2_ref, lng_ref, lnb_ref, p2_ref)


def _mlp(body, out_shape, h_st, agg, W1, b1, bng, bnb, p1, W2, b2, lng, lnb, p2):
    return pl.pallas_call(
        body,
        out_shape=jax.ShapeDtypeStruct(out_shape, jnp.float32),
    )(h_st, agg,
      W1, b1.reshape(1, D), bng.reshape(1, D), bnb.reshape(1, D),
      p1.reshape(1, 1),
      W2, b2.reshape(1, D), lng.reshape(1, D), lnb.reshape(1, D),
      p2.reshape(1, 1))


def kernel(x, edge_index, W1_0, b1_0, bng_0, bnb_0, p1_0, W2_0, b2_0, lng_0,
           lnb_0, p2_0, W1_1, b1_1, bng_1, bnb_1, p1_1, W2_1, b2_1, lng_1,
           lnb_1, p2_1):
    src2 = edge_index[0].reshape(NS, EPT)
    dst2 = edge_index[1].reshape(NS, EPT)
    tpad = jnp.arange(PAD_PER_TILE, dtype=jnp.int32)
    tidx = jnp.arange(NS, dtype=jnp.int32)
    pad_src = jnp.broadcast_to(N + (tpad % 8), (NS, PAD_PER_TILE)).astype(jnp.int32)
    pad_dst = ((tidx[:, None] * PAD_PER_TILE + tpad[None, :]) * 13 % N).astype(jnp.int32)
    srcp = jnp.concatenate([src2, pad_src], axis=1).reshape(-1)
    dstp = jnp.concatenate([dst2, pad_dst], axis=1).reshape(-1)

    xp = jnp.concatenate([x, jnp.zeros((NP - N, D), jnp.float32)], axis=0)
    x_st = jnp.stack([xp[:, :DH], xp[:, DH:]])
    zeros = jnp.zeros((N, DH), jnp.float32)

    agg = _sc_agg(x_st, srcp, dstp, zeros)
    h1_st = _mlp(_mlp_mid_body, (NC, NP, DH), x_st, agg,
                 W1_0, b1_0, bng_0, bnb_0, p1_0, W2_0, b2_0, lng_0, lnb_0, p2_0)
    agg = _sc_agg(h1_st, srcp, dstp, zeros)
    h2 = _mlp(_mlp_fin_body, (N, D), h1_st, agg,
              W1_1, b1_1, bng_1, bnb_1, p1_1, W2_1, b2_1, lng_1, lnb_1, p2_1)
    return h2


# speedup vs baseline: 9.2194x; 1.2441x over previous
import functools

import jax
import jax.numpy as jnp
from jax import lax
from jax.experimental import pallas as pl
from jax.experimental.pallas import tpu as pltpu
from jax.experimental.pallas import tpu_sc as plsc

N = 10000
E = 320000
D = 128
NP = N + 8

NC = 2
NS = 16
NW = NC * NS

CHUNK = 64
EPT = E // NW
EPT_PAD = 10240
NCHUNKS = EPT_PAD // CHUNK
PAD_PER_TILE = EPT_PAD - EPT
ROWS_PER_TILE = 624
TAIL_ROW0 = NS * ROWS_PER_TILE
TAIL_ROWS = N - TAIL_ROW0


def _sc_agg_body(h_hbm, src_hbm, dst_hbm, zeros_hbm, out_hbm,
                 src_v, dst_v, r0, r1, r2, agg_s, m0, m1, m2):
    c = lax.axis_index("c")
    s = lax.axis_index("s")
    rows = (r0, r1, r2)
    sems = (m0, m1, m2)

    ebase = (c * NS + s) * EPT_PAD
    row0 = s * ROWS_PER_TILE
    idx_src = pltpu.async_copy(src_hbm.at[pl.ds(ebase, EPT_PAD)], src_v, m0)
    idx_dst = pltpu.async_copy(dst_hbm.at[pl.ds(ebase, EPT_PAD)], dst_v, m1)
    zinit = pltpu.async_copy(zeros_hbm.at[pl.ds(row0, ROWS_PER_TILE)],
                             agg_s.at[pl.ds(row0, ROWS_PER_TILE)], m2)
    idx_src.wait()
    idx_dst.wait()
    zinit.wait()

    @pl.when(s == NS - 1)
    def _():
        pltpu.sync_copy(zeros_hbm.at[pl.ds(TAIL_ROW0, TAIL_ROWS)],
                        agg_s.at[pl.ds(TAIL_ROW0, TAIL_ROWS)])

    plsc.subcore_barrier()

    def sidx(k):
        return src_v.at[pl.ds(k * CHUNK, CHUNK)]

    def didx(k):
        return dst_v.at[pl.ds(k * CHUNK, CHUNK)]

    def issue_gather(k, b):
        pltpu.async_copy(h_hbm.at[sidx(k)], rows[b], sems[b])

    def wait_gather(k, b):
        pltpu.make_async_copy(h_hbm.at[sidx(k)], rows[b], sems[b]).wait()

    def issue_scatter(k, b):
        pltpu.async_copy(rows[b], agg_s.at[didx(k)], sems[b], add=True)

    def wait_scatter(k, b):
        pltpu.make_async_copy(rows[b], agg_s.at[didx(k)], sems[b]).wait()

    issue_gather(0, 0)
    issue_gather(1, 1)
    wait_gather(0, 0)
    issue_scatter(0, 0)
    issue_gather(2, 2)
    wait_gather(1, 1)
    issue_scatter(1, 1)
    wait_scatter(0, 0)
    issue_gather(3, 0)

    def body(i, carry):
        for j in range(3):
            k = 2 + 3 * i + j
            b = (2 + j) % 3
            wait_gather(k, b)
            issue_scatter(k, b)
            wait_scatter(k - 1, (b + 2) % 3)
            issue_gather(k + 2, (b + 2) % 3)
        return carry

    lax.fori_loop(0, (NCHUNKS - 4) // 3, body, 0)

    k = NCHUNKS - 2
    wait_gather(k, k % 3)
    issue_scatter(k, k % 3)
    wait_scatter(k - 1, (k - 1) % 3)
    wait_gather(k + 1, (k + 1) % 3)
    issue_scatter(k + 1, (k + 1) % 3)
    wait_scatter(k, k % 3)
    wait_scatter(k + 1, (k + 1) % 3)

    plsc.subcore_barrier()

    pltpu.sync_copy(agg_s.at[pl.ds(row0, ROWS_PER_TILE)],
                    out_hbm.at[c, pl.ds(row0, ROWS_PER_TILE)])

    @pl.when(s == NS - 1)
    def _():
        pltpu.sync_copy(agg_s.at[pl.ds(TAIL_ROW0, TAIL_ROWS)],
                        out_hbm.at[c, pl.ds(TAIL_ROW0, TAIL_ROWS)])


_sc_agg = functools.partial(
    pl.kernel,
    out_type=jax.ShapeDtypeStruct((NC, N, D), jnp.float32),
    mesh=plsc.VectorSubcoreMesh(core_axis_name="c", subcore_axis_name="s"),
    scratch_types=[
        pltpu.VMEM((EPT_PAD,), jnp.int32),
        pltpu.VMEM((EPT_PAD,), jnp.int32),
        pltpu.VMEM((CHUNK, D), jnp.float32),
        pltpu.VMEM((CHUNK, D), jnp.float32),
        pltpu.VMEM((CHUNK, D), jnp.float32),
        pltpu.VMEM_SHARED((N, D), jnp.float32),
        pltpu.SemaphoreType.DMA,
        pltpu.SemaphoreType.DMA,
        pltpu.SemaphoreType.DMA,
    ],
)(_sc_agg_body)


def _mlp_core(h_ref, a_ref, W1_ref, b1_ref, bng_ref, bnb_ref, p1_ref,
              W2_ref, b2_ref, lng_ref, lnb_ref, p2_ref):
    z = h_ref[:N, :] + a_ref[0] + a_ref[1]
    z = jnp.dot(z, W1_ref[...], preferred_element_type=jnp.float32) + b1_ref[...]
    mu = jnp.mean(z, axis=0, keepdims=True)
    var = jnp.mean((z - mu) ** 2, axis=0, keepdims=True)
    z = (z - mu) * lax.rsqrt(var + 1e-5) * bng_ref[...] + bnb_ref[...]
    z = jnp.where(z >= 0, z, p1_ref[0, 0] * z)
    z = jnp.dot(z, W2_ref[...], preferred_element_type=jnp.float32) + b2_ref[...]
    lmu = jnp.mean(z, axis=1, keepdims=True)
    lvar = jnp.mean((z - lmu) ** 2, axis=1, keepdims=True)
    z = (z - lmu) * lax.rsqrt(lvar + 1e-5) * lng_ref[...] + lnb_ref[...]
    return jnp.where(z >= 0, z, p2_ref[0, 0] * z)


def _mlp_mid_body(h_ref, a_ref, W1_ref, b1_ref, bng_ref, bnb_ref, p1_ref,
                  W2_ref, b2_ref, lng_ref, lnb_ref, p2_ref, out_ref):
    out_ref[:N, :] = _mlp_core(h_ref, a_ref, W1_ref, b1_ref, bng_ref, bnb_ref,
                               p1_ref, W2_ref, b2_ref, lng_ref, lnb_ref, p2_ref)
    out_ref[N:, :] = jnp.zeros((NP - N, D), jnp.float32)


def _mlp_fin_body(h_ref, a_ref, W1_ref, b1_ref, bng_ref, bnb_ref, p1_ref,
                  W2_ref, b2_ref, lng_ref, lnb_ref, p2_ref, out_ref):
    out_ref[...] = _mlp_core(h_ref, a_ref, W1_ref, b1_ref, bng_ref, bnb_ref,
                             p1_ref, W2_ref, b2_ref, lng_ref, lnb_ref, p2_ref)


def _mlp(body, out_rows, h_pad, agg, W1, b1, bng, bnb, p1, W2, b2, lng, lnb, p2):
    return pl.pallas_call(
        body,
        out_shape=jax.ShapeDtypeStruct((out_rows, D), jnp.float32),
    )(h_pad, agg,
      W1, b1.reshape(1, D), bng.reshape(1, D), bnb.reshape(1, D),
      p1.reshape(1, 1),
      W2, b2.reshape(1, D), lng.reshape(1, D), lnb.reshape(1, D),
      p2.reshape(1, 1))


def kernel(x, edge_index, W1_0, b1_0, bng_0, bnb_0, p1_0, W2_0, b2_0, lng_0,
           lnb_0, p2_0, W1_1, b1_1, bng_1, bnb_1, p1_1, W2_1, b2_1, lng_1,
           lnb_1, p2_1):
    src2 = edge_index[0].reshape(NW, EPT)
    dst2 = edge_index[1].reshape(NW, EPT)
    tpad = jnp.arange(PAD_PER_TILE, dtype=jnp.int32)
    tidx = jnp.arange(NW, dtype=jnp.int32)
    pad_src = jnp.broadcast_to(N + (tpad % 8), (NW, PAD_PER_TILE)).astype(jnp.int32)
    pad_dst = ((tidx[:, None] * PAD_PER_TILE + tpad[None, :]) * 13 % N).astype(jnp.int32)
    srcp = jnp.concatenate([src2, pad_src], axis=1).reshape(-1)
    dstp = jnp.concatenate([dst2, pad_dst], axis=1).reshape(-1)

    x_pad = jnp.concatenate([x, jnp.zeros((NP - N, D), jnp.float32)], axis=0)
    zeros = jnp.zeros((N, D), jnp.float32)

    agg = _sc_agg(x_pad, srcp, dstp, zeros)
    h1 = _mlp(_mlp_mid_body, NP, x_pad, agg,
              W1_0, b1_0, bng_0, bnb_0, p1_0, W2_0, b2_0, lng_0, lnb_0, p2_0)
    agg = _sc_agg(h1, srcp, dstp, zeros)
    h2 = _mlp(_mlp_fin_body, N, h1, agg,
              W1_1, b1_1, bng_1, bnb_1, p1_1, W2_1, b2_1, lng_1, lnb_1, p2_1)
    return h2


# speedup vs baseline: 12.2557x; 1.3293x over previous
import functools

import jax
import jax.numpy as jnp
from jax import lax
from jax.experimental import pallas as pl
from jax.experimental.pallas import tpu as pltpu
from jax.experimental.pallas import tpu_sc as plsc

N = 10000
E = 320000
D = 128
NP = N + 8

NC = 2
NS = 16
NW = NC * NS

CHUNK = 80
EPT = E // NW
NCHUNKS = EPT // CHUNK
ROWS_PER_TILE = 624
TAIL_ROW0 = NS * ROWS_PER_TILE
TAIL_ROWS = N - TAIL_ROW0


def _sc_agg_body(h_hbm, src_hbm, dst_hbm, zeros_hbm, out_hbm,
                 src_v, dst_v, r0, r1, r2, agg_s, m0, m1, m2):
    c = lax.axis_index("c")
    s = lax.axis_index("s")
    rows = (r0, r1, r2)
    sems = (m0, m1, m2)

    ebase = (c * NS + s) * EPT
    row0 = s * ROWS_PER_TILE
    idx_src = pltpu.async_copy(src_hbm.at[pl.ds(ebase, EPT)], src_v, m0)
    idx_dst = pltpu.async_copy(dst_hbm.at[pl.ds(ebase, EPT)], dst_v, m1)
    zinit = pltpu.async_copy(zeros_hbm.at[pl.ds(row0, ROWS_PER_TILE)],
                             agg_s.at[pl.ds(row0, ROWS_PER_TILE)], m2)
    idx_src.wait()
    idx_dst.wait()
    zinit.wait()

    @pl.when(s == NS - 1)
    def _():
        pltpu.sync_copy(zeros_hbm.at[pl.ds(TAIL_ROW0, TAIL_ROWS)],
                        agg_s.at[pl.ds(TAIL_ROW0, TAIL_ROWS)])

    plsc.subcore_barrier()

    def sidx(k):
        return src_v.at[pl.ds(k * CHUNK, CHUNK)]

    def didx(k):
        return dst_v.at[pl.ds(k * CHUNK, CHUNK)]

    def issue_gather(k, b):
        pltpu.async_copy(h_hbm.at[sidx(k)], rows[b], sems[b])

    def wait_gather(k, b):
        pltpu.make_async_copy(h_hbm.at[sidx(k)], rows[b], sems[b]).wait()

    def issue_scatter(k, b):
        pltpu.async_copy(rows[b], agg_s.at[didx(k)], sems[b], add=True)

    def wait_scatter(k, b):
        pltpu.make_async_copy(rows[b], agg_s.at[didx(k)], sems[b]).wait()

    issue_gather(0, 0)
    issue_gather(1, 1)
    wait_gather(0, 0)
    issue_scatter(0, 0)
    issue_gather(2, 2)
    wait_gather(1, 1)
    issue_scatter(1, 1)
    wait_scatter(0, 0)
    issue_gather(3, 0)

    def body(i, carry):
        for j in range(3):
            k = 2 + 3 * i + j
            b = (2 + j) % 3
            wait_gather(k, b)
            issue_scatter(k, b)
            wait_scatter(k - 1, (b + 2) % 3)
            issue_gather(k + 2, (b + 2) % 3)
        return carry

    lax.fori_loop(0, (NCHUNKS - 5) // 3, body, 0)

    k = NCHUNKS - 3
    wait_gather(k, k % 3)
    issue_scatter(k, k % 3)
    wait_scatter(k - 1, (k - 1) % 3)
    issue_gather(k + 2, (k + 2) % 3)
    wait_gather(k + 1, (k + 1) % 3)
    issue_scatter(k + 1, (k + 1) % 3)
    wait_scatter(k, k % 3)
    wait_gather(k + 2, (k + 2) % 3)
    issue_scatter(k + 2, (k + 2) % 3)
    wait_scatter(k + 1, (k + 1) % 3)
    wait_scatter(k + 2, (k + 2) % 3)

    plsc.subcore_barrier()

    pltpu.sync_copy(agg_s.at[pl.ds(row0, ROWS_PER_TILE)],
                    out_hbm.at[c, pl.ds(row0, ROWS_PER_TILE)])

    @pl.when(s == NS - 1)
    def _():
        pltpu.sync_copy(agg_s.at[pl.ds(TAIL_ROW0, TAIL_ROWS)],
                        out_hbm.at[c, pl.ds(TAIL_ROW0, TAIL_ROWS)])


_sc_agg = functools.partial(
    pl.kernel,
    out_type=jax.ShapeDtypeStruct((NC, N, D), jnp.float32),
    mesh=plsc.VectorSubcoreMesh(core_axis_name="c", subcore_axis_name="s"),
    scratch_types=[
        pltpu.VMEM((EPT,), jnp.int32),
        pltpu.VMEM((EPT,), jnp.int32),
        pltpu.VMEM((CHUNK, D), jnp.float32),
        pltpu.VMEM((CHUNK, D), jnp.float32),
        pltpu.VMEM((CHUNK, D), jnp.float32),
        pltpu.VMEM_SHARED((N, D), jnp.float32),
        pltpu.SemaphoreType.DMA,
        pltpu.SemaphoreType.DMA,
        pltpu.SemaphoreType.DMA,
    ],
)(_sc_agg_body)


def _mlp_core(h_ref, a_ref, W1_ref, b1_ref, bng_ref, bnb_ref, p1_ref,
              W2_ref, b2_ref, lng_ref, lnb_ref, p2_ref):
    z = h_ref[...] + a_ref[0] + a_ref[1]
    z = jnp.dot(z, W1_ref[...], preferred_element_type=jnp.float32) + b1_ref[...]
    mu = jnp.mean(z, axis=0, keepdims=True)
    var = jnp.mean((z - mu) ** 2, axis=0, keepdims=True)
    z = (z - mu) * lax.rsqrt(var + 1e-5) * bng_ref[...] + bnb_ref[...]
    z = jnp.where(z >= 0, z, p1_ref[0, 0] * z)
    z = jnp.dot(z, W2_ref[...], preferred_element_type=jnp.float32) + b2_ref[...]
    lmu = jnp.mean(z, axis=1, keepdims=True)
    lvar = jnp.mean((z - lmu) ** 2, axis=1, keepdims=True)
    z = (z - lmu) * lax.rsqrt(lvar + 1e-5) * lng_ref[...] + lnb_ref[...]
    return jnp.where(z >= 0, z, p2_ref[0, 0] * z)


def _mlp_body(h_ref, a_ref, W1_ref, b1_ref, bng_ref, bnb_ref, p1_ref,
              W2_ref, b2_ref, lng_ref, lnb_ref, p2_ref, out_ref):
    out_ref[...] = _mlp_core(h_ref, a_ref, W1_ref, b1_ref, bng_ref, bnb_ref,
                             p1_ref, W2_ref, b2_ref, lng_ref, lnb_ref, p2_ref)


def _mlp(h, agg, W1, b1, bng, bnb, p1, W2, b2, lng, lnb, p2):
    return pl.pallas_call(
        _mlp_body,
        out_shape=jax.ShapeDtypeStruct((N, D), jnp.float32),
    )(h, agg,
      W1, b1.reshape(1, D), bng.reshape(1, D), bnb.reshape(1, D),
      p1.reshape(1, 1),
      W2, b2.reshape(1, D), lng.reshape(1, D), lnb.reshape(1, D),
      p2.reshape(1, 1))


def kernel(x, edge_index, W1_0, b1_0, bng_0, bnb_0, p1_0, W2_0, b2_0, lng_0,
           lnb_0, p2_0, W1_1, b1_1, bng_1, bnb_1, p1_1, W2_1, b2_1, lng_1,
           lnb_1, p2_1):
    src = edge_index[0]
    dst = edge_index[1]
    zeros = jnp.zeros((N, D), jnp.float32)

    agg = _sc_agg(x, src, dst, zeros)
    h1 = _mlp(x, agg,
              W1_0, b1_0, bng_0, bnb_0, p1_0, W2_0, b2_0, lng_0, lnb_0, p2_0)
    agg = _sc_agg(h1, src, dst, zeros)
    h2 = _mlp(h1, agg,
              W1_1, b1_1, bng_1, bnb_1, p1_1, W2_1, b2_1, lng_1, lnb_1, p2_1)
    return h2


# speedup vs baseline: 12.3590x; 1.0084x over previous
import functools

import jax
import jax.numpy as jnp
from jax import lax
from jax.experimental import pallas as pl
from jax.experimental.pallas import tpu as pltpu
from jax.experimental.pallas import tpu_sc as plsc

N = 10000
E = 320000
D = 128
NP = N + 8

NC = 2
NS = 16
NW = NC * NS

CHUNK = 80
EPT = E // NW
NCHUNKS = EPT // CHUNK
ROWS_PER_TILE = 624
TAIL_ROW0 = NS * ROWS_PER_TILE
TAIL_ROWS = N - TAIL_ROW0


def _sc_agg_body(h_hbm, src_hbm, dst_hbm, zeros_hbm, out_hbm,
                 src_v, dst_v, r0, r1, r2, agg_s, m0, m1, m2):
    c = lax.axis_index("c")
    s = lax.axis_index("s")
    rows = (r0, r1, r2)
    sems = (m0, m1, m2)

    ebase = (c * NS + s) * EPT
    row0 = s * ROWS_PER_TILE
    idx_src = pltpu.async_copy(src_hbm.at[pl.ds(ebase, EPT)], src_v, m0)
    idx_dst = pltpu.async_copy(dst_hbm.at[pl.ds(ebase, EPT)], dst_v, m1)
    zinit = pltpu.async_copy(zeros_hbm.at[pl.ds(row0, ROWS_PER_TILE)],
                             agg_s.at[pl.ds(row0, ROWS_PER_TILE)], m2)
    def sidx(k):
        return src_v.at[pl.ds(k * CHUNK, CHUNK)]

    def didx(k):
        return dst_v.at[pl.ds(k * CHUNK, CHUNK)]

    def issue_gather(k, b):
        pltpu.async_copy(h_hbm.at[sidx(k)], rows[b], sems[b])

    def wait_gather(k, b):
        pltpu.make_async_copy(h_hbm.at[sidx(k)], rows[b], sems[b]).wait()

    def issue_scatter(k, b):
        pltpu.async_copy(rows[b], agg_s.at[didx(k)], sems[b], add=True)

    def wait_scatter(k, b):
        pltpu.make_async_copy(rows[b], agg_s.at[didx(k)], sems[b]).wait()

    idx_src.wait()
    idx_dst.wait()
    issue_gather(0, 0)
    issue_gather(1, 1)
    zinit.wait()

    @pl.when(s == NS - 1)
    def _():
        pltpu.sync_copy(zeros_hbm.at[pl.ds(TAIL_ROW0, TAIL_ROWS)],
                        agg_s.at[pl.ds(TAIL_ROW0, TAIL_ROWS)])

    plsc.subcore_barrier()
    wait_gather(0, 0)
    issue_scatter(0, 0)
    issue_gather(2, 2)
    wait_gather(1, 1)
    issue_scatter(1, 1)
    wait_scatter(0, 0)
    issue_gather(3, 0)

    def body(i, carry):
        for j in range(3):
            k = 2 + 3 * i + j
            b = (2 + j) % 3
            wait_gather(k, b)
            issue_scatter(k, b)
            wait_scatter(k - 1, (b + 2) % 3)
            issue_gather(k + 2, (b + 2) % 3)
        return carry

    lax.fori_loop(0, (NCHUNKS - 5) // 3, body, 0)

    k = NCHUNKS - 3
    wait_gather(k, k % 3)
    issue_scatter(k, k % 3)
    wait_scatter(k - 1, (k - 1) % 3)
    issue_gather(k + 2, (k + 2) % 3)
    wait_gather(k + 1, (k + 1) % 3)
    issue_scatter(k + 1, (k + 1) % 3)
    wait_scatter(k, k % 3)
    wait_gather(k + 2, (k + 2) % 3)
    issue_scatter(k + 2, (k + 2) % 3)
    wait_scatter(k + 1, (k + 1) % 3)
    wait_scatter(k + 2, (k + 2) % 3)

    plsc.subcore_barrier()

    pltpu.sync_copy(agg_s.at[pl.ds(row0, ROWS_PER_TILE)],
                    out_hbm.at[c, pl.ds(row0, ROWS_PER_TILE)])

    @pl.when(s == NS - 1)
    def _():
        pltpu.sync_copy(agg_s.at[pl.ds(TAIL_ROW0, TAIL_ROWS)],
                        out_hbm.at[c, pl.ds(TAIL_ROW0, TAIL_ROWS)])


_sc_agg = functools.partial(
    pl.kernel,
    out_type=jax.ShapeDtypeStruct((NC, N, D), jnp.float32),
    mesh=plsc.VectorSubcoreMesh(core_axis_name="c", subcore_axis_name="s"),
    scratch_types=[
        pltpu.VMEM((EPT,), jnp.int32),
        pltpu.VMEM((EPT,), jnp.int32),
        pltpu.VMEM((CHUNK, D), jnp.float32),
        pltpu.VMEM((CHUNK, D), jnp.float32),
        pltpu.VMEM((CHUNK, D), jnp.float32),
        pltpu.VMEM_SHARED((N, D), jnp.float32),
        pltpu.SemaphoreType.DMA,
        pltpu.SemaphoreType.DMA,
        pltpu.SemaphoreType.DMA,
    ],
)(_sc_agg_body)


def _mlp_core(h_ref, a_ref, W1_ref, b1_ref, bng_ref, bnb_ref, p1_ref,
              W2_ref, b2_ref, lng_ref, lnb_ref, p2_ref):
    z = h_ref[...] + a_ref[0] + a_ref[1]
    z = jnp.dot(z, W1_ref[...], preferred_element_type=jnp.float32) + b1_ref[...]
    mu = jnp.mean(z, axis=0, keepdims=True)
    var = jnp.mean((z - mu) ** 2, axis=0, keepdims=True)
    z = (z - mu) * lax.rsqrt(var + 1e-5) * bng_ref[...] + bnb_ref[...]
    z = jnp.where(z >= 0, z, p1_ref[0, 0] * z)
    z = jnp.dot(z, W2_ref[...], preferred_element_type=jnp.float32) + b2_ref[...]
    lmu = jnp.mean(z, axis=1, keepdims=True)
    lvar = jnp.mean((z - lmu) ** 2, axis=1, keepdims=True)
    z = (z - lmu) * lax.rsqrt(lvar + 1e-5) * lng_ref[...] + lnb_ref[...]
    return jnp.where(z >= 0, z, p2_ref[0, 0] * z)


def _mlp_body(h_ref, a_ref, W1_ref, b1_ref, bng_ref, bnb_ref, p1_ref,
              W2_ref, b2_ref, lng_ref, lnb_ref, p2_ref, out_ref):
    out_ref[...] = _mlp_core(h_ref, a_ref, W1_ref, b1_ref, bng_ref, bnb_ref,
                             p1_ref, W2_ref, b2_ref, lng_ref, lnb_ref, p2_ref)


def _mlp(h, agg, W1, b1, bng, bnb, p1, W2, b2, lng, lnb, p2):
    return pl.pallas_call(
        _mlp_body,
        out_shape=jax.ShapeDtypeStruct((N, D), jnp.float32),
    )(h, agg,
      W1, b1.reshape(1, D), bng.reshape(1, D), bnb.reshape(1, D),
      p1.reshape(1, 1),
      W2, b2.reshape(1, D), lng.reshape(1, D), lnb.reshape(1, D),
      p2.reshape(1, 1))


def kernel(x, edge_index, W1_0, b1_0, bng_0, bnb_0, p1_0, W2_0, b2_0, lng_0,
           lnb_0, p2_0, W1_1, b1_1, bng_1, bnb_1, p1_1, W2_1, b2_1, lng_1,
           lnb_1, p2_1):
    src = edge_index[0]
    dst = edge_index[1]
    zeros = jnp.zeros((N, D), jnp.float32)

    agg = _sc_agg(x, src, dst, zeros)
    h1 = _mlp(x, agg,
              W1_0, b1_0, bng_0, bnb_0, p1_0, W2_0, b2_0, lng_0, lnb_0, p2_0)
    agg = _sc_agg(h1, src, dst, zeros)
    h2 = _mlp(h1, agg,
              W1_1, b1_1, bng_1, bnb_1, p1_1, W2_1, b2_1, lng_1, lnb_1, p2_1)
    return h2


# speedup vs baseline: 12.5256x; 1.0135x over previous
import functools

import jax
import jax.numpy as jnp
from jax import lax
from jax.experimental import pallas as pl
from jax.experimental.pallas import tpu as pltpu
from jax.experimental.pallas import tpu_sc as plsc

N = 10000
E = 320000
D = 128
NP = N + 8

NC = 2
NS = 16
NW = NC * NS

CHUNK = 80
EPT = E // NW
NCHUNKS = EPT // CHUNK
ROWS_PER_TILE = 624
TAIL_ROW0 = NS * ROWS_PER_TILE
TAIL_ROWS = N - TAIL_ROW0


def _sc_agg_body(h_hbm, src_hbm, dst_hbm, zeros_hbm, out_hbm,
                 src_v, dst_v, r0, r1, r2, agg_s, m0, m1, m2):
    c = lax.axis_index("c")
    s = lax.axis_index("s")
    rows = (r0, r1, r2)
    sems = (m0, m1, m2)

    ebase = (c * NS + s) * EPT
    row0 = s * ROWS_PER_TILE
    idx_src = pltpu.async_copy(src_hbm.at[pl.ds(ebase, EPT)], src_v, m0)
    idx_dst = pltpu.async_copy(dst_hbm.at[pl.ds(ebase, EPT)], dst_v, m1)
    @pl.when(c == 0)
    def _():
        pltpu.async_copy(h_hbm.at[pl.ds(row0, ROWS_PER_TILE)],
                         agg_s.at[pl.ds(row0, ROWS_PER_TILE)], m2)

    @pl.when(c == 1)
    def _():
        pltpu.async_copy(zeros_hbm.at[pl.ds(row0, ROWS_PER_TILE)],
                         agg_s.at[pl.ds(row0, ROWS_PER_TILE)], m2)
    zinit = pltpu.make_async_copy(zeros_hbm.at[pl.ds(row0, ROWS_PER_TILE)],
                                  agg_s.at[pl.ds(row0, ROWS_PER_TILE)], m2)
    def sidx(k):
        return src_v.at[pl.ds(k * CHUNK, CHUNK)]

    def didx(k):
        return dst_v.at[pl.ds(k * CHUNK, CHUNK)]

    def issue_gather(k, b):
        pltpu.async_copy(h_hbm.at[sidx(k)], rows[b], sems[b])

    def wait_gather(k, b):
        pltpu.make_async_copy(h_hbm.at[sidx(k)], rows[b], sems[b]).wait()

    def issue_scatter(k, b):
        pltpu.async_copy(rows[b], agg_s.at[didx(k)], sems[b], add=True)

    def wait_scatter(k, b):
        pltpu.make_async_copy(rows[b], agg_s.at[didx(k)], sems[b]).wait()

    idx_src.wait()
    idx_dst.wait()
    issue_gather(0, 0)
    issue_gather(1, 1)
    zinit.wait()

    @pl.when((s == NS - 1) & (c == 0))
    def _():
        pltpu.sync_copy(h_hbm.at[pl.ds(TAIL_ROW0, TAIL_ROWS)],
                        agg_s.at[pl.ds(TAIL_ROW0, TAIL_ROWS)])

    @pl.when((s == NS - 1) & (c == 1))
    def _():
        pltpu.sync_copy(zeros_hbm.at[pl.ds(TAIL_ROW0, TAIL_ROWS)],
                        agg_s.at[pl.ds(TAIL_ROW0, TAIL_ROWS)])

    plsc.subcore_barrier()
    wait_gather(0, 0)
    issue_scatter(0, 0)
    issue_gather(2, 2)
    wait_gather(1, 1)
    issue_scatter(1, 1)
    wait_scatter(0, 0)
    issue_gather(3, 0)

    def body(i, carry):
        for j in range(3):
            k = 2 + 3 * i + j
            b = (2 + j) % 3
            wait_gather(k, b)
            issue_scatter(k, b)
            wait_scatter(k - 1, (b + 2) % 3)
            issue_gather(k + 2, (b + 2) % 3)
        return carry

    lax.fori_loop(0, (NCHUNKS - 5) // 3, body, 0)

    k = NCHUNKS - 3
    wait_gather(k, k % 3)
    issue_scatter(k, k % 3)
    wait_scatter(k - 1, (k - 1) % 3)
    issue_gather(k + 2, (k + 2) % 3)
    wait_gather(k + 1, (k + 1) % 3)
    issue_scatter(k + 1, (k + 1) % 3)
    wait_scatter(k, k % 3)
    wait_gather(k + 2, (k + 2) % 3)
    issue_scatter(k + 2, (k + 2) % 3)
    wait_scatter(k + 1, (k + 1) % 3)
    wait_scatter(k + 2, (k + 2) % 3)

    plsc.subcore_barrier()

    pltpu.sync_copy(agg_s.at[pl.ds(row0, ROWS_PER_TILE)],
                    out_hbm.at[c, pl.ds(row0, ROWS_PER_TILE)])

    @pl.when(s == NS - 1)
    def _():
        pltpu.sync_copy(agg_s.at[pl.ds(TAIL_ROW0, TAIL_ROWS)],
                        out_hbm.at[c, pl.ds(TAIL_ROW0, TAIL_ROWS)])


_sc_agg = functools.partial(
    pl.kernel,
    out_type=jax.ShapeDtypeStruct((NC, N, D), jnp.float32),
    mesh=plsc.VectorSubcoreMesh(core_axis_name="c", subcore_axis_name="s"),
    scratch_types=[
        pltpu.VMEM((EPT,), jnp.int32),
        pltpu.VMEM((EPT,), jnp.int32),
        pltpu.VMEM((CHUNK, D), jnp.float32),
        pltpu.VMEM((CHUNK, D), jnp.float32),
        pltpu.VMEM((CHUNK, D), jnp.float32),
        pltpu.VMEM_SHARED((N, D), jnp.float32),
        pltpu.SemaphoreType.DMA,
        pltpu.SemaphoreType.DMA,
        pltpu.SemaphoreType.DMA,
    ],
)(_sc_agg_body)


def _mlp_core(a_ref, W1_ref, b1_ref, bng_ref, bnb_ref, p1_ref,
              W2_ref, b2_ref, lng_ref, lnb_ref, p2_ref):
    z = a_ref[0] + a_ref[1]
    z = jnp.dot(z, W1_ref[...], preferred_element_type=jnp.float32) + b1_ref[...]
    mu = jnp.mean(z, axis=0, keepdims=True)
    var = jnp.mean((z - mu) ** 2, axis=0, keepdims=True)
    z = (z - mu) * lax.rsqrt(var + 1e-5) * bng_ref[...] + bnb_ref[...]
    z = jnp.where(z >= 0, z, p1_ref[0, 0] * z)
    z = jnp.dot(z, W2_ref[...], preferred_element_type=jnp.float32) + b2_ref[...]
    lmu = jnp.mean(z, axis=1, keepdims=True)
    lvar = jnp.mean((z - lmu) ** 2, axis=1, keepdims=True)
    z = (z - lmu) * lax.rsqrt(lvar + 1e-5) * lng_ref[...] + lnb_ref[...]
    return jnp.where(z >= 0, z, p2_ref[0, 0] * z)


def _mlp_body(a_ref, W1_ref, b1_ref, bng_ref, bnb_ref, p1_ref,
              W2_ref, b2_ref, lng_ref, lnb_ref, p2_ref, out_ref):
    out_ref[...] = _mlp_core(a_ref, W1_ref, b1_ref, bng_ref, bnb_ref,
                             p1_ref, W2_ref, b2_ref, lng_ref, lnb_ref, p2_ref)


def _mlp(agg, W1, b1, bng, bnb, p1, W2, b2, lng, lnb, p2):
    return pl.pallas_call(
        _mlp_body,
        out_shape=jax.ShapeDtypeStruct((N, D), jnp.float32),
    )(agg,
      W1, b1.reshape(1, D), bng.reshape(1, D), bnb.reshape(1, D),
      p1.reshape(1, 1),
      W2, b2.reshape(1, D), lng.reshape(1, D), lnb.reshape(1, D),
      p2.reshape(1, 1))


def kernel(x, edge_index, W1_0, b1_0, bng_0, bnb_0, p1_0, W2_0, b2_0, lng_0,
           lnb_0, p2_0, W1_1, b1_1, bng_1, bnb_1, p1_1, W2_1, b2_1, lng_1,
           lnb_1, p2_1):
    src = edge_index[0]
    dst = edge_index[1]
    zeros = jnp.zeros((N, D), jnp.float32)

    agg = _sc_agg(x, src, dst, zeros)
    h1 = _mlp(agg,
              W1_0, b1_0, bng_0, bnb_0, p1_0, W2_0, b2_0, lng_0, lnb_0, p2_0)
    agg = _sc_agg(h1, src, dst, zeros)
    h2 = _mlp(agg,
              W1_1, b1_1, bng_1, bnb_1, p1_1, W2_1, b2_1, lng_1, lnb_1, p2_1)
    return h2


# speedup vs baseline: 12.9280x; 1.0321x over previous
import functools

import jax
import jax.numpy as jnp
from jax import lax
from jax.experimental import pallas as pl
from jax.experimental.pallas import tpu as pltpu
from jax.experimental.pallas import tpu_sc as plsc

N = 10000
E = 320000
D = 128
NP = N + 8

NC = 2
NS = 16
NW = NC * NS

CHUNK = 80
EPT = E // NW
NCHUNKS = EPT // CHUNK
ROWS_PER_TILE = 624
TAIL_ROW0 = NS * ROWS_PER_TILE
TAIL_ROWS = N - TAIL_ROW0


def _sc_agg_body(h_hbm, src_hbm, dst_hbm, zeros_hbm, out_hbm,
                 src_v, dst_v, r0, r1, r2, agg_s, m0, m1, m2):
    c = lax.axis_index("c")
    s = lax.axis_index("s")
    rows = (r0, r1, r2)
    sems = (m0, m1, m2)

    ebase = (c * NS + s) * EPT
    row0 = s * ROWS_PER_TILE
    idx_src = pltpu.async_copy(src_hbm.at[pl.ds(ebase, EPT)], src_v, m0)
    idx_dst = pltpu.async_copy(dst_hbm.at[pl.ds(ebase, EPT)], dst_v, m1)
    @pl.when(c == 0)
    def _():
        pltpu.async_copy(h_hbm.at[pl.ds(row0, ROWS_PER_TILE)],
                         agg_s.at[pl.ds(row0, ROWS_PER_TILE)], m2)

    @pl.when(c == 1)
    def _():
        pltpu.async_copy(zeros_hbm.at[pl.ds(row0, ROWS_PER_TILE)],
                         agg_s.at[pl.ds(row0, ROWS_PER_TILE)], m2)
    zinit = pltpu.make_async_copy(zeros_hbm.at[pl.ds(row0, ROWS_PER_TILE)],
                                  agg_s.at[pl.ds(row0, ROWS_PER_TILE)], m2)
    def sidx(k):
        return src_v.at[pl.ds(k * CHUNK, CHUNK)]

    def didx(k):
        return dst_v.at[pl.ds(k * CHUNK, CHUNK)]

    def issue_gather(k, b):
        pltpu.async_copy(h_hbm.at[sidx(k)], rows[b], sems[b])

    def wait_gather(k, b):
        pltpu.make_async_copy(h_hbm.at[sidx(k)], rows[b], sems[b]).wait()

    def issue_scatter(k, b):
        pltpu.async_copy(rows[b], agg_s.at[didx(k)], sems[b], add=True)

    def wait_scatter(k, b):
        pltpu.make_async_copy(rows[b], agg_s.at[didx(k)], sems[b]).wait()

    idx_src.wait()
    idx_dst.wait()
    issue_gather(0, 0)
    issue_gather(1, 1)
    zinit.wait()

    @pl.when((s == NS - 1) & (c == 0))
    def _():
        pltpu.sync_copy(h_hbm.at[pl.ds(TAIL_ROW0, TAIL_ROWS)],
                        agg_s.at[pl.ds(TAIL_ROW0, TAIL_ROWS)])

    @pl.when((s == NS - 1) & (c == 1))
    def _():
        pltpu.sync_copy(zeros_hbm.at[pl.ds(TAIL_ROW0, TAIL_ROWS)],
                        agg_s.at[pl.ds(TAIL_ROW0, TAIL_ROWS)])

    plsc.subcore_barrier()
    wait_gather(0, 0)
    issue_scatter(0, 0)
    issue_gather(2, 2)
    wait_scatter(0, 0)
    issue_gather(3, 0)
    wait_gather(1, 1)
    issue_scatter(1, 1)

    def body(i, carry):
        for j in range(3):
            k = 2 + 3 * i + j
            b = (2 + j) % 3
            wait_scatter(k - 1, (b + 2) % 3)
            issue_gather(k + 2, (b + 2) % 3)
            wait_gather(k, b)
            issue_scatter(k, b)
        return carry

    lax.fori_loop(0, (NCHUNKS - 5) // 3, body, 0)

    k = NCHUNKS - 3
    wait_scatter(k - 1, (k - 1) % 3)
    issue_gather(k + 2, (k + 2) % 3)
    wait_gather(k, k % 3)
    issue_scatter(k, k % 3)
    wait_gather(k + 1, (k + 1) % 3)
    issue_scatter(k + 1, (k + 1) % 3)
    wait_scatter(k, k % 3)
    wait_gather(k + 2, (k + 2) % 3)
    issue_scatter(k + 2, (k + 2) % 3)
    wait_scatter(k + 1, (k + 1) % 3)
    wait_scatter(k + 2, (k + 2) % 3)

    plsc.subcore_barrier()

    pltpu.sync_copy(agg_s.at[pl.ds(row0, ROWS_PER_TILE)],
                    out_hbm.at[c, pl.ds(row0, ROWS_PER_TILE)])

    @pl.when(s == NS - 1)
    def _():
        pltpu.sync_copy(agg_s.at[pl.ds(TAIL_ROW0, TAIL_ROWS)],
                        out_hbm.at[c, pl.ds(TAIL_ROW0, TAIL_ROWS)])


_sc_agg = functools.partial(
    pl.kernel,
    out_type=jax.ShapeDtypeStruct((NC, N, D), jnp.float32),
    mesh=plsc.VectorSubcoreMesh(core_axis_name="c", subcore_axis_name="s"),
    scratch_types=[
        pltpu.VMEM((EPT,), jnp.int32),
        pltpu.VMEM((EPT,), jnp.int32),
        pltpu.VMEM((CHUNK, D), jnp.float32),
        pltpu.VMEM((CHUNK, D), jnp.float32),
        pltpu.VMEM((CHUNK, D), jnp.float32),
        pltpu.VMEM_SHARED((N, D), jnp.float32),
        pltpu.SemaphoreType.DMA,
        pltpu.SemaphoreType.DMA,
        pltpu.SemaphoreType.DMA,
    ],
)(_sc_agg_body)


def _mlp_core(a_ref, W1_ref, b1_ref, bng_ref, bnb_ref, p1_ref,
              W2_ref, b2_ref, lng_ref, lnb_ref, p2_ref):
    z = a_ref[0] + a_ref[1]
    z = jnp.dot(z, W1_ref[...], preferred_element_type=jnp.float32) + b1_ref[...]
    mu = jnp.mean(z, axis=0, keepdims=True)
    var = jnp.mean((z - mu) ** 2, axis=0, keepdims=True)
    z = (z - mu) * lax.rsqrt(var + 1e-5) * bng_ref[...] + bnb_ref[...]
    z = jnp.where(z >= 0, z, p1_ref[0, 0] * z)
    z = jnp.dot(z, W2_ref[...], preferred_element_type=jnp.float32) + b2_ref[...]
    lmu = jnp.mean(z, axis=1, keepdims=True)
    lvar = jnp.mean((z - lmu) ** 2, axis=1, keepdims=True)
    z = (z - lmu) * lax.rsqrt(lvar + 1e-5) * lng_ref[...] + lnb_ref[...]
    return jnp.where(z >= 0, z, p2_ref[0, 0] * z)


def _mlp_body(a_ref, W1_ref, b1_ref, bng_ref, bnb_ref, p1_ref,
              W2_ref, b2_ref, lng_ref, lnb_ref, p2_ref, out_ref):
    out_ref[...] = _mlp_core(a_ref, W1_ref, b1_ref, bng_ref, bnb_ref,
                             p1_ref, W2_ref, b2_ref, lng_ref, lnb_ref, p2_ref)


def _mlp(agg, W1, b1, bng, bnb, p1, W2, b2, lng, lnb, p2):
    return pl.pallas_call(
        _mlp_body,
        out_shape=jax.ShapeDtypeStruct((N, D), jnp.float32),
    )(agg,
      W1, b1.reshape(1, D), bng.reshape(1, D), bnb.reshape(1, D),
      p1.reshape(1, 1),
      W2, b2.reshape(1, D), lng.reshape(1, D), lnb.reshape(1, D),
      p2.reshape(1, 1))


def kernel(x, edge_index, W1_0, b1_0, bng_0, bnb_0, p1_0, W2_0, b2_0, lng_0,
           lnb_0, p2_0, W1_1, b1_1, bng_1, bnb_1, p1_1, W2_1, b2_1, lng_1,
           lnb_1, p2_1):
    src = edge_index[0]
    dst = edge_index[1]
    zeros = jnp.zeros((N, D), jnp.float32)

    agg = _sc_agg(x, src, dst, zeros)
    h1 = _mlp(agg,
              W1_0, b1_0, bng_0, bnb_0, p1_0, W2_0, b2_0, lng_0, lnb_0, p2_0)
    agg = _sc_agg(h1, src, dst, zeros)
    h2 = _mlp(agg,
              W1_1, b1_1, bng_1, bnb_1, p1_1, W2_1, b2_1, lng_1, lnb_1, p2_1)
    return h2


# speedup vs baseline: 12.9395x; 1.0009x over previous
import functools

import jax
import jax.numpy as jnp
from jax import lax
from jax.experimental import pallas as pl
from jax.experimental.pallas import tpu as pltpu
from jax.experimental.pallas import tpu_sc as plsc

N = 10000
E = 320000
D = 128

NC = 2
NS = 16
NW = NC * NS

CHUNK = 80
EPT = E // NW
NCHUNKS = EPT // CHUNK
ROWS_PER_TILE = 624
TAIL_ROW0 = NS * ROWS_PER_TILE
TAIL_ROWS = N - TAIL_ROW0


def _sc_agg_body(h_hbm, src_hbm, dst_hbm, zeros_hbm, out_hbm,
                 src_v, dst_v, r0, r1, r2, agg_s, m0, m1, m2):
    c = lax.axis_index("c")
    s = lax.axis_index("s")
    rows = (r0, r1, r2)
    sems = (m0, m1, m2)

    ebase = (c * NS + s) * EPT
    row0 = s * ROWS_PER_TILE
    idx_src = pltpu.async_copy(src_hbm.at[pl.ds(ebase, EPT)], src_v, m0)
    idx_dst = pltpu.async_copy(dst_hbm.at[pl.ds(ebase, EPT)], dst_v, m1)
    @pl.when(c == 0)
    def _():
        pltpu.async_copy(h_hbm.at[pl.ds(row0, ROWS_PER_TILE)],
                         agg_s.at[pl.ds(row0, ROWS_PER_TILE)], m2)

    @pl.when(c == 1)
    def _():
        pltpu.async_copy(zeros_hbm.at[pl.ds(row0, ROWS_PER_TILE)],
                         agg_s.at[pl.ds(row0, ROWS_PER_TILE)], m2)
    zinit = pltpu.make_async_copy(zeros_hbm.at[pl.ds(row0, ROWS_PER_TILE)],
                                  agg_s.at[pl.ds(row0, ROWS_PER_TILE)], m2)
    def sidx(k):
        return src_v.at[pl.ds(k * CHUNK, CHUNK)]

    def didx(k):
        return dst_v.at[pl.ds(k * CHUNK, CHUNK)]

    def issue_gather(k, b):
        pltpu.async_copy(h_hbm.at[sidx(k)], rows[b], sems[b])

    def wait_gather(k, b):
        pltpu.make_async_copy(h_hbm.at[sidx(k)], rows[b], sems[b]).wait()

    def issue_scatter(k, b):
        pltpu.async_copy(rows[b], agg_s.at[didx(k)], sems[b], add=True)

    def wait_scatter(k, b):
        pltpu.make_async_copy(rows[b], agg_s.at[didx(k)], sems[b]).wait()

    idx_src.wait()
    idx_dst.wait()
    issue_gather(0, 0)
    issue_gather(1, 1)
    zinit.wait()

    @pl.when((s == NS - 1) & (c == 0))
    def _():
        pltpu.sync_copy(h_hbm.at[pl.ds(TAIL_ROW0, TAIL_ROWS)],
                        agg_s.at[pl.ds(TAIL_ROW0, TAIL_ROWS)])

    @pl.when((s == NS - 1) & (c == 1))
    def _():
        pltpu.sync_copy(zeros_hbm.at[pl.ds(TAIL_ROW0, TAIL_ROWS)],
                        agg_s.at[pl.ds(TAIL_ROW0, TAIL_ROWS)])

    plsc.subcore_barrier()
    wait_gather(0, 0)
    issue_scatter(0, 0)
    issue_gather(2, 2)
    wait_scatter(0, 0)
    issue_gather(3, 0)
    wait_gather(1, 1)
    issue_scatter(1, 1)

    def body(i, carry):
        for j in range(3):
            k = 2 + 3 * i + j
            b = (2 + j) % 3
            wait_scatter(k - 1, (b + 2) % 3)
            issue_gather(k + 2, (b + 2) % 3)
            wait_gather(k, b)
            issue_scatter(k, b)
        return carry

    lax.fori_loop(0, (NCHUNKS - 5) // 3, body, 0)

    k = NCHUNKS - 3
    wait_scatter(k - 1, (k - 1) % 3)
    issue_gather(k + 2, (k + 2) % 3)
    wait_gather(k, k % 3)
    issue_scatter(k, k % 3)
    wait_gather(k + 1, (k + 1) % 3)
    issue_scatter(k + 1, (k + 1) % 3)
    wait_scatter(k, k % 3)
    wait_gather(k + 2, (k + 2) % 3)
    issue_scatter(k + 2, (k + 2) % 3)
    wait_scatter(k + 1, (k + 1) % 3)
    wait_scatter(k + 2, (k + 2) % 3)

    plsc.subcore_barrier()

    pltpu.sync_copy(agg_s.at[pl.ds(row0, ROWS_PER_TILE)],
                    out_hbm.at[c, pl.ds(row0, ROWS_PER_TILE)])

    @pl.when(s == NS - 1)
    def _():
        pltpu.sync_copy(agg_s.at[pl.ds(TAIL_ROW0, TAIL_ROWS)],
                        out_hbm.at[c, pl.ds(TAIL_ROW0, TAIL_ROWS)])


_sc_agg = functools.partial(
    pl.kernel,
    out_type=jax.ShapeDtypeStruct((NC, N, D), jnp.float32),
    mesh=plsc.VectorSubcoreMesh(core_axis_name="c", subcore_axis_name="s"),
    scratch_types=[
        pltpu.VMEM((EPT,), jnp.int32),
        pltpu.VMEM((EPT,), jnp.int32),
        pltpu.VMEM((CHUNK, D), jnp.float32),
        pltpu.VMEM((CHUNK, D), jnp.float32),
        pltpu.VMEM((CHUNK, D), jnp.float32),
        pltpu.VMEM_SHARED((N, D), jnp.float32),
        pltpu.SemaphoreType.DMA,
        pltpu.SemaphoreType.DMA,
        pltpu.SemaphoreType.DMA,
    ],
)(_sc_agg_body)


def _mlp_core(a_ref, W1_ref, b1_ref, bng_ref, bnb_ref, p1_ref,
              W2_ref, b2_ref, lng_ref, lnb_ref, p2_ref):
    z = a_ref[0] + a_ref[1]
    z = jnp.dot(z, W1_ref[...], preferred_element_type=jnp.float32) + b1_ref[...]
    mu = jnp.mean(z, axis=0, keepdims=True)
    var = jnp.mean((z - mu) ** 2, axis=0, keepdims=True)
    z = (z - mu) * lax.rsqrt(var + 1e-5) * bng_ref[...] + bnb_ref[...]
    z = jnp.where(z >= 0, z, p1_ref[0, 0] * z)
    z = jnp.dot(z, W2_ref[...], preferred_element_type=jnp.float32) + b2_ref[...]
    lmu = jnp.mean(z, axis=1, keepdims=True)
    lvar = jnp.mean((z - lmu) ** 2, axis=1, keepdims=True)
    z = (z - lmu) * lax.rsqrt(lvar + 1e-5) * lng_ref[...] + lnb_ref[...]
    return jnp.where(z >= 0, z, p2_ref[0, 0] * z)


def _mlp_body(a_ref, W1_ref, b1_ref, bng_ref, bnb_ref, p1_ref,
              W2_ref, b2_ref, lng_ref, lnb_ref, p2_ref, out_ref):
    out_ref[...] = _mlp_core(a_ref, W1_ref, b1_ref, bng_ref, bnb_ref,
                             p1_ref, W2_ref, b2_ref, lng_ref, lnb_ref, p2_ref)


def _mlp(agg, W1, b1, bng, bnb, p1, W2, b2, lng, lnb, p2):
    return pl.pallas_call(
        _mlp_body,
        out_shape=jax.ShapeDtypeStruct((N, D), jnp.float32),
    )(agg,
      W1, b1.reshape(1, D), bng.reshape(1, D), bnb.reshape(1, D),
      p1.reshape(1, 1),
      W2, b2.reshape(1, D), lng.reshape(1, D), lnb.reshape(1, D),
      p2.reshape(1, 1))


def kernel(x, edge_index, W1_0, b1_0, bng_0, bnb_0, p1_0, W2_0, b2_0, lng_0,
           lnb_0, p2_0, W1_1, b1_1, bng_1, bnb_1, p1_1, W2_1, b2_1, lng_1,
           lnb_1, p2_1):
    src = edge_index[0]
    dst = edge_index[1]
    zeros = jnp.zeros((N, D), jnp.float32)

    agg = _sc_agg(x, src, dst, zeros)
    h1 = _mlp(agg,
              W1_0, b1_0, bng_0, bnb_0, p1_0, W2_0, b2_0, lng_0, lnb_0, p2_0)
    agg = _sc_agg(h1, src, dst, zeros)
    h2 = _mlp(agg,
              W1_1, b1_1, bng_1, bnb_1, p1_1, W2_1, b2_1, lng_1, lnb_1, p2_1)
    return h2
